# baseline (speedup 1.0000x reference)
import functools

import jax
import jax.numpy as jnp
from jax import lax
from jax.experimental import pallas as pl
from jax.experimental.pallas import tpu as pltpu
from jax.experimental.pallas import tpu_sc as plsc

S = 2048
B = 16
DP = 16
NPATCH = 128
DM = 768
DH = 1536
NH = 12
NL = 2
E = 8
TK = 2
D_HEAD = DM // NH
T = NPATCH * B
A_RAW = T * TK
TILE = 128
A_PAD = A_RAW + E * TILE
IMP_F = 0.01
LOAD_F = 0.01
EPS = 1e-5


def _mm(a, b, bias=None, res=None, act=None, ln=None, row_affine=None, bm=256):
    m, k = a.shape
    n = b.shape[1]
    grid = (m // bm,)
    in_specs = [
        pl.BlockSpec((bm, k), lambda i: (i, 0)),
        pl.BlockSpec((k, n), lambda i: (0, 0)),
    ]
    args = [a, b]
    if bias is not None:
        in_specs.append(pl.BlockSpec((1, n), lambda i: (0, 0)))
        args.append(bias.reshape(1, n))
    if res is not None:
        in_specs.append(pl.BlockSpec((bm, n), lambda i: (i, 0)))
        args.append(res)
    if row_affine is not None:
        in_specs.append(pl.BlockSpec((bm, 1), lambda i: (i, 0)))
        in_specs.append(pl.BlockSpec((bm, 1), lambda i: (i, 0)))
        args.extend([row_affine[0], row_affine[1]])
    if ln is not None:
        in_specs.append(pl.BlockSpec((1, n), lambda i: (0, 0)))
        in_specs.append(pl.BlockSpec((1, n), lambda i: (0, 0)))
        args.extend([ln[0].reshape(1, n), ln[1].reshape(1, n)])

    def body(*refs):
        it = iter(refs)
        a_ref = next(it)
        b_ref = next(it)
        y = jnp.dot(a_ref[...], b_ref[...], preferred_element_type=jnp.float32)
        if bias is not None:
            y = y + next(it)[...]
        if res is not None:
            y = y + next(it)[...]
        if act == "relu":
            y = jnp.maximum(y, 0.0)
        if row_affine is not None:
            rs = next(it)[...]
            ra = next(it)[...]
            y = y * rs + ra
        if ln is not None:
            g_ref = next(it)[...]
            be_ref = next(it)[...]
            mu = jnp.mean(y, -1, keepdims=True)
            var = jnp.mean((y - mu) * (y - mu), -1, keepdims=True)
            y = (y - mu) / jnp.sqrt(var + 1e-5) * g_ref + be_ref
        o_ref = next(it)
        o_ref[...] = y

    return pl.pallas_call(
        body,
        grid=grid,
        in_specs=in_specs,
        out_specs=pl.BlockSpec((bm, n), lambda i: (i, 0)),
        out_shape=jax.ShapeDtypeStruct((m, n), jnp.float32),
    )(*args)


def _instnorm_kernel(x_ref, xn_ref, mu_ref, sd_ref):
    x = x_ref[...]
    mu = jnp.mean(x, 0, keepdims=True)
    xc = x - mu
    sd = jnp.sqrt(jnp.mean(xc * xc, 0, keepdims=True))
    xn_ref[...] = xc / (sd + EPS)
    mu_ref[...] = mu
    sd_ref[...] = sd


def _instnorm(x2):
    return pl.pallas_call(
        _instnorm_kernel,
        grid=(1,),
        in_specs=[pl.BlockSpec((S, B), lambda i: (0, 0))],
        out_specs=[
            pl.BlockSpec((S, B), lambda i: (0, 0)),
            pl.BlockSpec((1, B), lambda i: (0, 0)),
            pl.BlockSpec((1, B), lambda i: (0, 0)),
        ],
        out_shape=[
            jax.ShapeDtypeStruct((S, B), jnp.float32),
            jax.ShapeDtypeStruct((1, B), jnp.float32),
            jax.ShapeDtypeStruct((1, B), jnp.float32),
        ],
    )(x2)


def _attn_kernel(q_ref, k_ref, v_ref, o_ref):
    q = q_ref[0]
    k = k_ref[0]
    v = v_ref[0]
    s = lax.dot_general(q, k, (((1,), (1,)), ((), ())),
                        preferred_element_type=jnp.float32) * 0.125
    mx = jnp.max(s, -1, keepdims=True)
    p = jnp.exp(s - mx)
    p = p / jnp.sum(p, -1, keepdims=True)
    o_ref[0] = jnp.dot(p, v, preferred_element_type=jnp.float32)


def _attention(q, k, v):
    bh = q.shape[0]
    spec = pl.BlockSpec((1, NPATCH, D_HEAD), lambda i: (i, 0, 0))
    return pl.pallas_call(
        _attn_kernel,
        grid=(bh,),
        in_specs=[spec, spec, spec],
        out_specs=spec,
        out_shape=jax.ShapeDtypeStruct((bh, NPATCH, D_HEAD), jnp.float32),
    )(q, k, v)


def _router_kernel(h_ref, wg_ref, idx_ref, w_ref, imp_ref, load_ref):
    logits = jnp.dot(h_ref[...], wg_ref[...], preferred_element_type=jnp.float32)
    bm = logits.shape[0]
    iota = lax.broadcasted_iota(jnp.int32, (bm, E), 1)
    m1 = jnp.max(logits, 1, keepdims=True)
    a1 = jnp.min(jnp.where(logits == m1, iota, E), 1, keepdims=True)
    oh1 = iota == a1
    masked = jnp.where(oh1, -jnp.inf, logits)
    m2 = jnp.max(masked, 1, keepdims=True)
    a2 = jnp.min(jnp.where(masked == m2, iota, E), 1, keepdims=True)
    oh2 = iota == a2
    x = jnp.exp(m2 - m1)
    d = 1.0 + x
    g1 = 1.0 / d
    g2 = x / d
    gates = jnp.where(oh1, g1, 0.0) + jnp.where(oh2, g2, 0.0)
    idx_ref[...] = jnp.concatenate([a1, a2], axis=1)
    w_ref[...] = jnp.concatenate([g1, g2], axis=1)

    @pl.when(pl.program_id(0) == 0)
    def _():
        imp_ref[...] = jnp.zeros_like(imp_ref)
        load_ref[...] = jnp.zeros_like(load_ref)

    imp_ref[...] += jnp.sum(gates, 0, keepdims=True)
    load_ref[...] += jnp.sum((gates > 0.0).astype(jnp.float32), 0, keepdims=True)


def _router(h2, wg, bm=256):
    return pl.pallas_call(
        _router_kernel,
        grid=(T // bm,),
        in_specs=[
            pl.BlockSpec((bm, DM), lambda i: (i, 0)),
            pl.BlockSpec((DM, E), lambda i: (0, 0)),
        ],
        out_specs=[
            pl.BlockSpec((bm, TK), lambda i: (i, 0)),
            pl.BlockSpec((bm, TK), lambda i: (i, 0)),
            pl.BlockSpec((1, E), lambda i: (0, 0)),
            pl.BlockSpec((1, E), lambda i: (0, 0)),
        ],
        out_shape=[
            jax.ShapeDtypeStruct((T, TK), jnp.int32),
            jax.ShapeDtypeStruct((T, TK), jnp.float32),
            jax.ShapeDtypeStruct((1, E), jnp.float32),
            jax.ShapeDtypeStruct((1, E), jnp.float32),
        ],
    )(h2, wg)


def _gm1_kernel(gid_ref, x_ref, w_ref, b_ref, o_ref):
    y = jnp.dot(x_ref[...], w_ref[0], preferred_element_type=jnp.float32)
    o_ref[...] = jnp.maximum(y + b_ref[0], 0.0)


def _gm1(tile_gid, xg, we1, be1):
    grid_spec = pltpu.PrefetchScalarGridSpec(
        num_scalar_prefetch=1,
        grid=(A_PAD // TILE,),
        in_specs=[
            pl.BlockSpec((TILE, DM), lambda i, g: (i, 0)),
            pl.BlockSpec((1, DM, DH), lambda i, g: (g[i], 0, 0)),
            pl.BlockSpec((1, 1, DH), lambda i, g: (g[i], 0, 0)),
        ],
        out_specs=pl.BlockSpec((TILE, DH), lambda i, g: (i, 0)),
    )
    return pl.pallas_call(
        _gm1_kernel,
        grid_spec=grid_spec,
        out_shape=jax.ShapeDtypeStruct((A_PAD, DH), jnp.float32),
    )(tile_gid, xg, we1, be1.reshape(E, 1, DH))


def _gm2_kernel(gid_ref, y_ref, w_ref, b_ref, g_ref, o_ref):
    y = jnp.dot(y_ref[...], w_ref[0], preferred_element_type=jnp.float32)
    o_ref[...] = (y + b_ref[0]) * g_ref[...]


def _gm2(tile_gid, y1, we2, be2, gate_pad):
    grid_spec = pltpu.PrefetchScalarGridSpec(
        num_scalar_prefetch=1,
        grid=(A_PAD // TILE,),
        in_specs=[
            pl.BlockSpec((TILE, DH), lambda i, g: (i, 0)),
            pl.BlockSpec((1, DH, DM), lambda i, g: (g[i], 0, 0)),
            pl.BlockSpec((1, 1, DM), lambda i, g: (g[i], 0, 0)),
            pl.BlockSpec((TILE, 1), lambda i, g: (i, 0)),
        ],
        out_specs=pl.BlockSpec((TILE, DM), lambda i, g: (i, 0)),
    )
    return pl.pallas_call(
        _gm2_kernel,
        grid_spec=grid_spec,
        out_shape=jax.ShapeDtypeStruct((A_PAD, DM), jnp.float32),
    )(tile_gid, y1, we2, be2.reshape(E, 1, DM), gate_pad)


def _combine_kernel(yg_ref, h_ref, g_ref, b_ref, o_ref):
    y = yg_ref[:, 0, :] + yg_ref[:, 1, :] + h_ref[...]
    mu = jnp.mean(y, -1, keepdims=True)
    var = jnp.mean((y - mu) * (y - mu), -1, keepdims=True)
    o_ref[...] = (y - mu) / jnp.sqrt(var + 1e-5) * g_ref[...] + b_ref[...]


def _combine_ln(yg3, h2, g, b, bm=256):
    return pl.pallas_call(
        _combine_kernel,
        grid=(T // bm,),
        in_specs=[
            pl.BlockSpec((bm, TK, DM), lambda i: (i, 0, 0)),
            pl.BlockSpec((bm, DM), lambda i: (i, 0)),
            pl.BlockSpec((1, DM), lambda i: (0, 0)),
            pl.BlockSpec((1, DM), lambda i: (0, 0)),
        ],
        out_specs=pl.BlockSpec((bm, DM), lambda i: (i, 0)),
        out_shape=jax.ShapeDtypeStruct((T, DM), jnp.float32),
    )(yg3, h2, g.reshape(1, DM), b.reshape(1, DM))


@functools.lru_cache(maxsize=None)
def _make_sc_gather(v_rows, d, n_out):
    info = plsc.get_sparse_core_info()
    nw = info.num_cores * info.num_subcores
    assert n_out % (8 * nw) == 0 and d % info.num_lanes == 0
    bpw = n_out // nw
    mesh = plsc.VectorSubcoreMesh(core_axis_name="c", subcore_axis_name="s")

    @functools.partial(
        pl.kernel,
        mesh=mesh,
        out_type=jax.ShapeDtypeStruct((n_out, d), jnp.float32),
        scratch_types=[
            pltpu.VMEM((bpw,), jnp.int32),
            pltpu.VMEM((bpw, d), jnp.float32),
            pltpu.SemaphoreType.DMA,
        ],
    )
    def k(table_hbm, idx_hbm, out_hbm, idx_v, rows_v, sem):
        wid = lax.axis_index("s") * info.num_cores + lax.axis_index("c")
        base = wid * bpw
        pltpu.sync_copy(idx_hbm.at[pl.ds(base, bpw)], idx_v)
        pltpu.async_copy(table_hbm.at[idx_v], rows_v, sem).wait()
        pltpu.sync_copy(rows_v, out_hbm.at[pl.ds(base, bpw)])

    return k


def _sc_gather_dispatch(table, idx):
    return _make_sc_gather(T, DM, A_PAD)(table, idx)


def _sc_gather_combine(table, idx):
    return _make_sc_gather(A_PAD, DM, A_RAW)(table, idx)


def _route_tables(idx, w):
    flat_e = idx.reshape(A_RAW)
    flat_t = jnp.arange(A_RAW, dtype=jnp.int32) // TK
    flat_g = w.reshape(A_RAW)
    order = jnp.argsort(flat_e, stable=True)
    se = flat_e[order]
    st = flat_t[order]
    sg = flat_g[order]
    counts = jnp.sum(flat_e[:, None] == jnp.arange(E, dtype=jnp.int32)[None, :],
                     axis=0, dtype=jnp.int32)
    pc = ((counts + TILE - 1) // TILE) * TILE
    zero = jnp.zeros((1,), jnp.int32)
    pstart = jnp.concatenate([zero, jnp.cumsum(pc)[:-1]])
    offs = jnp.concatenate([zero, jnp.cumsum(counts)[:-1]])
    r = jnp.arange(A_RAW, dtype=jnp.int32) - offs[se]
    p = pstart[se] + r
    src = jnp.zeros((A_PAD,), jnp.int32).at[p].set(st)
    gpad = jnp.zeros((A_PAD,), jnp.float32).at[p].set(sg)
    posflat = jnp.zeros((A_RAW,), jnp.int32).at[order].set(p)
    tile_rows = jnp.arange(A_PAD // TILE, dtype=jnp.int32) * TILE
    tile_gid = jnp.clip(
        jnp.searchsorted(pstart, tile_rows, side="right") - 1, 0, E - 1
    ).astype(jnp.int32)
    return src, gpad.reshape(A_PAD, 1), tile_gid, posflat


def _cv_sq(v):
    return jnp.var(v) / (jnp.mean(v) ** 2 + 1e-10)


def _pos_enc():
    pos = jnp.arange(NPATCH, dtype=jnp.float32)[:, None]
    even = jnp.arange(0, DM, 2, dtype=jnp.float32)
    odd = jnp.arange(1, DM, 2, dtype=jnp.float32)
    enc = jnp.zeros((NPATCH, DM), jnp.float32)
    enc = enc.at[:, 0::2].set(jnp.sin(pos / jnp.power(10000.0, even / DM)))
    enc = enc.at[:, 1::2].set(jnp.cos(pos / jnp.power(10000.0, odd / DM)))
    return enc


def kernel(x, W_emb, b_emb, Wq, bq, Wk, bk, Wv, bv, Wo, bo, ln1_g, ln1_b,
           ln2_g, ln2_b, W_gate, We1, be1, We2, be2, W_fc, b_fc, W_out, b_out):
    x2 = x.reshape(S, B)
    xn, mu, sd = _instnorm(x2)
    patches = xn.reshape(NPATCH, DP, B).transpose(0, 2, 1).reshape(T, DP)
    pe_full = jnp.repeat(_pos_enc(), B, axis=0)
    h2 = _mm(patches, W_emb, bias=b_emb, res=pe_full)

    aux = jnp.asarray(0.0, jnp.float32)
    for l in range(NL):
        wqkv = jnp.concatenate([Wq[l], Wk[l], Wv[l]], axis=1)
        bqkv = jnp.concatenate([bq[l], bk[l], bv[l]], axis=0)
        qkv = _mm(h2, wqkv, bias=bqkv)
        qkv5 = qkv.reshape(NPATCH, B, 3, NH, D_HEAD).transpose(2, 1, 3, 0, 4)
        qkv5 = qkv5.reshape(3, B * NH, NPATCH, D_HEAD)
        o = _attention(qkv5[0], qkv5[1], qkv5[2])
        o = o.reshape(B, NH, NPATCH, D_HEAD).transpose(2, 0, 1, 3).reshape(T, DM)
        h2 = _mm(o, Wo[l], bias=bo[l], res=h2, ln=(ln1_g[l], ln1_b[l]))

        idx, w, imp, loads = _router(h2, W_gate[l])
        aux = aux + IMP_F * _cv_sq(imp[0]) + LOAD_F * _cv_sq(loads[0])

        src, gpad, tile_gid, posflat = _route_tables(idx, w)
        xg = _sc_gather_dispatch(h2, src)
        y1 = _gm1(tile_gid, xg, We1[l], be1[l])
        y2 = _gm2(tile_gid, y1, We2[l], be2[l], gpad)
        yg = _sc_gather_combine(y2, posflat)
        h2 = _combine_ln(yg.reshape(T, TK, DM), h2, ln2_g[l], ln2_b[l])

    dxp = _mm(h2, W_fc, bias=b_fc, act="relu")
    dxb = dxp.reshape(NPATCH, B, DP).transpose(1, 0, 2).reshape(B, NPATCH * DP)
    rs = (sd + EPS).reshape(B, 1)
    ra = mu.reshape(B, 1)
    y = _mm(dxb, W_out, bias=b_out, row_affine=(rs, ra), bm=B)
    out = y.transpose(1, 0).reshape(S, B, 1)
    return out, aux


# speedup vs baseline: 1.1944x; 1.1944x over previous
import functools

import jax
import jax.numpy as jnp
from jax import lax
from jax.experimental import pallas as pl
from jax.experimental.pallas import tpu as pltpu
from jax.experimental.pallas import tpu_sc as plsc

S = 2048
B = 16
DP = 16
NPATCH = 128
DM = 768
DH = 1536
NH = 12
NL = 2
E = 8
TK = 2
D_HEAD = DM // NH
T = NPATCH * B
A_RAW = T * TK
TILE = 128
A_PAD = A_RAW + E * TILE
IMP_F = 0.01
LOAD_F = 0.01
EPS = 1e-5


def _mm(a, b, bias=None, res=None, act=None, ln=None, row_affine=None, bm=256):
    m, k = a.shape
    n = b.shape[1]
    grid = (m // bm,)
    in_specs = [
        pl.BlockSpec((bm, k), lambda i: (i, 0)),
        pl.BlockSpec((k, n), lambda i: (0, 0)),
    ]
    args = [a, b]
    if bias is not None:
        in_specs.append(pl.BlockSpec((1, n), lambda i: (0, 0)))
        args.append(bias.reshape(1, n))
    if res is not None:
        in_specs.append(pl.BlockSpec((bm, n), lambda i: (i, 0)))
        args.append(res)
    if row_affine is not None:
        in_specs.append(pl.BlockSpec((bm, 1), lambda i: (i, 0)))
        in_specs.append(pl.BlockSpec((bm, 1), lambda i: (i, 0)))
        args.extend([row_affine[0], row_affine[1]])
    if ln is not None:
        in_specs.append(pl.BlockSpec((1, n), lambda i: (0, 0)))
        in_specs.append(pl.BlockSpec((1, n), lambda i: (0, 0)))
        args.extend([ln[0].reshape(1, n), ln[1].reshape(1, n)])

    def body(*refs):
        it = iter(refs)
        a_ref = next(it)
        b_ref = next(it)
        y = jnp.dot(a_ref[...], b_ref[...], preferred_element_type=jnp.float32)
        if bias is not None:
            y = y + next(it)[...]
        if res is not None:
            y = y + next(it)[...]
        if act == "relu":
            y = jnp.maximum(y, 0.0)
        if row_affine is not None:
            rs = next(it)[...]
            ra = next(it)[...]
            y = y * rs + ra
        if ln is not None:
            g_ref = next(it)[...]
            be_ref = next(it)[...]
            mu = jnp.mean(y, -1, keepdims=True)
            var = jnp.mean((y - mu) * (y - mu), -1, keepdims=True)
            y = (y - mu) / jnp.sqrt(var + 1e-5) * g_ref + be_ref
        o_ref = next(it)
        o_ref[...] = y

    return pl.pallas_call(
        body,
        grid=grid,
        in_specs=in_specs,
        out_specs=pl.BlockSpec((bm, n), lambda i: (i, 0)),
        out_shape=jax.ShapeDtypeStruct((m, n), jnp.float32),
    )(*args)


def _instnorm_kernel(x_ref, xn_ref, mu_ref, sd_ref):
    x = x_ref[...]
    mu = jnp.mean(x, 0, keepdims=True)
    xc = x - mu
    sd = jnp.sqrt(jnp.mean(xc * xc, 0, keepdims=True))
    xn_ref[...] = xc / (sd + EPS)
    mu_ref[...] = mu
    sd_ref[...] = sd


def _instnorm(x2):
    return pl.pallas_call(
        _instnorm_kernel,
        grid=(1,),
        in_specs=[pl.BlockSpec((S, B), lambda i: (0, 0))],
        out_specs=[
            pl.BlockSpec((S, B), lambda i: (0, 0)),
            pl.BlockSpec((1, B), lambda i: (0, 0)),
            pl.BlockSpec((1, B), lambda i: (0, 0)),
        ],
        out_shape=[
            jax.ShapeDtypeStruct((S, B), jnp.float32),
            jax.ShapeDtypeStruct((1, B), jnp.float32),
            jax.ShapeDtypeStruct((1, B), jnp.float32),
        ],
    )(x2)


def _attn_kernel(q_ref, k_ref, v_ref, o_ref):
    q = q_ref[0]
    k = k_ref[0]
    v = v_ref[0]
    s = lax.dot_general(q, k, (((1,), (1,)), ((), ())),
                        preferred_element_type=jnp.float32) * 0.125
    mx = jnp.max(s, -1, keepdims=True)
    p = jnp.exp(s - mx)
    p = p / jnp.sum(p, -1, keepdims=True)
    o_ref[0] = jnp.dot(p, v, preferred_element_type=jnp.float32)


def _attention(q, k, v):
    bh = q.shape[0]
    spec = pl.BlockSpec((1, NPATCH, D_HEAD), lambda i: (i, 0, 0))
    return pl.pallas_call(
        _attn_kernel,
        grid=(bh,),
        in_specs=[spec, spec, spec],
        out_specs=spec,
        out_shape=jax.ShapeDtypeStruct((bh, NPATCH, D_HEAD), jnp.float32),
    )(q, k, v)


def _router_kernel(h_ref, wg_ref, idx_ref, w_ref, imp_ref, load_ref):
    logits = jnp.dot(h_ref[...], wg_ref[...], preferred_element_type=jnp.float32)
    bm = logits.shape[0]
    iota = lax.broadcasted_iota(jnp.int32, (bm, E), 1)
    m1 = jnp.max(logits, 1, keepdims=True)
    a1 = jnp.min(jnp.where(logits == m1, iota, E), 1, keepdims=True)
    oh1 = iota == a1
    masked = jnp.where(oh1, -jnp.inf, logits)
    m2 = jnp.max(masked, 1, keepdims=True)
    a2 = jnp.min(jnp.where(masked == m2, iota, E), 1, keepdims=True)
    oh2 = iota == a2
    x = jnp.exp(m2 - m1)
    d = 1.0 + x
    g1 = 1.0 / d
    g2 = x / d
    gates = jnp.where(oh1, g1, 0.0) + jnp.where(oh2, g2, 0.0)
    idx_ref[...] = jnp.concatenate([a1, a2], axis=1)
    w_ref[...] = jnp.concatenate([g1, g2], axis=1)

    @pl.when(pl.program_id(0) == 0)
    def _():
        imp_ref[...] = jnp.zeros_like(imp_ref)
        load_ref[...] = jnp.zeros_like(load_ref)

    imp_ref[...] += jnp.sum(gates, 0, keepdims=True)
    load_ref[...] += jnp.sum((gates > 0.0).astype(jnp.float32), 0, keepdims=True)


def _router(h2, wg, bm=256):
    return pl.pallas_call(
        _router_kernel,
        grid=(T // bm,),
        in_specs=[
            pl.BlockSpec((bm, DM), lambda i: (i, 0)),
            pl.BlockSpec((DM, E), lambda i: (0, 0)),
        ],
        out_specs=[
            pl.BlockSpec((bm, TK), lambda i: (i, 0)),
            pl.BlockSpec((bm, TK), lambda i: (i, 0)),
            pl.BlockSpec((1, E), lambda i: (0, 0)),
            pl.BlockSpec((1, E), lambda i: (0, 0)),
        ],
        out_shape=[
            jax.ShapeDtypeStruct((T, TK), jnp.int32),
            jax.ShapeDtypeStruct((T, TK), jnp.float32),
            jax.ShapeDtypeStruct((1, E), jnp.float32),
            jax.ShapeDtypeStruct((1, E), jnp.float32),
        ],
    )(h2, wg)


def _gm1_kernel(gid_ref, x_ref, w_ref, b_ref, o_ref):
    y = jnp.dot(x_ref[...], w_ref[0], preferred_element_type=jnp.float32)
    o_ref[...] = jnp.maximum(y + b_ref[0], 0.0)


def _gm1(tile_gid, xg, we1, be1):
    grid_spec = pltpu.PrefetchScalarGridSpec(
        num_scalar_prefetch=1,
        grid=(A_PAD // TILE,),
        in_specs=[
            pl.BlockSpec((TILE, DM), lambda i, g: (i, 0)),
            pl.BlockSpec((1, DM, DH), lambda i, g: (g[i], 0, 0)),
            pl.BlockSpec((1, 1, DH), lambda i, g: (g[i], 0, 0)),
        ],
        out_specs=pl.BlockSpec((TILE, DH), lambda i, g: (i, 0)),
    )
    return pl.pallas_call(
        _gm1_kernel,
        grid_spec=grid_spec,
        out_shape=jax.ShapeDtypeStruct((A_PAD, DH), jnp.float32),
    )(tile_gid, xg, we1, be1.reshape(E, 1, DH))


def _gm2_kernel(gid_ref, y_ref, w_ref, b_ref, o_ref):
    y = jnp.dot(y_ref[...], w_ref[0], preferred_element_type=jnp.float32)
    o_ref[...] = y + b_ref[0]


def _gm2(tile_gid, y1, we2, be2):
    grid_spec = pltpu.PrefetchScalarGridSpec(
        num_scalar_prefetch=1,
        grid=(A_PAD // TILE,),
        in_specs=[
            pl.BlockSpec((TILE, DH), lambda i, g: (i, 0)),
            pl.BlockSpec((1, DH, DM), lambda i, g: (g[i], 0, 0)),
            pl.BlockSpec((1, 1, DM), lambda i, g: (g[i], 0, 0)),
        ],
        out_specs=pl.BlockSpec((TILE, DM), lambda i, g: (i, 0)),
    )
    return pl.pallas_call(
        _gm2_kernel,
        grid_spec=grid_spec,
        out_shape=jax.ShapeDtypeStruct((A_PAD, DM), jnp.float32),
    )(tile_gid, y1, we2, be2.reshape(E, 1, DM))


def _combine_kernel(yg_ref, w_ref, h_ref, g_ref, b_ref, o_ref):
    w = w_ref[...]
    y = (yg_ref[:, 0, :] * w[:, 0:1] + yg_ref[:, 1, :] * w[:, 1:2]
         + h_ref[...])
    mu = jnp.mean(y, -1, keepdims=True)
    var = jnp.mean((y - mu) * (y - mu), -1, keepdims=True)
    o_ref[...] = (y - mu) / jnp.sqrt(var + 1e-5) * g_ref[...] + b_ref[...]


def _combine_ln(yg3, w, h2, g, b, bm=256):
    return pl.pallas_call(
        _combine_kernel,
        grid=(T // bm,),
        in_specs=[
            pl.BlockSpec((bm, TK, DM), lambda i: (i, 0, 0)),
            pl.BlockSpec((bm, TK), lambda i: (i, 0)),
            pl.BlockSpec((bm, DM), lambda i: (i, 0)),
            pl.BlockSpec((1, DM), lambda i: (0, 0)),
            pl.BlockSpec((1, DM), lambda i: (0, 0)),
        ],
        out_specs=pl.BlockSpec((bm, DM), lambda i: (i, 0)),
        out_shape=jax.ShapeDtypeStruct((T, DM), jnp.float32),
    )(yg3, w, h2, g.reshape(1, DM), b.reshape(1, DM))


@functools.lru_cache(maxsize=None)
def _make_sc_gather(v_rows, d, n_out):
    info = plsc.get_sparse_core_info()
    nw = info.num_cores * info.num_subcores
    assert n_out % (8 * nw) == 0 and d % info.num_lanes == 0
    bpw = n_out // nw
    mesh = plsc.VectorSubcoreMesh(core_axis_name="c", subcore_axis_name="s")

    @functools.partial(
        pl.kernel,
        mesh=mesh,
        out_type=jax.ShapeDtypeStruct((n_out, d), jnp.float32),
        scratch_types=[
            pltpu.VMEM((bpw,), jnp.int32),
            pltpu.VMEM((bpw, d), jnp.float32),
            pltpu.SemaphoreType.DMA,
        ],
    )
    def k(table_hbm, idx_hbm, out_hbm, idx_v, rows_v, sem):
        wid = lax.axis_index("s") * info.num_cores + lax.axis_index("c")
        base = wid * bpw
        pltpu.sync_copy(idx_hbm.at[pl.ds(base, bpw)], idx_v)
        pltpu.async_copy(table_hbm.at[idx_v], rows_v, sem).wait()
        pltpu.sync_copy(rows_v, out_hbm.at[pl.ds(base, bpw)])

    return k


def _sc_gather_combine(table, idx):
    return _make_sc_gather(A_PAD, DM, A_RAW)(table, idx)


@functools.lru_cache(maxsize=None)
def _make_sc_dispatch():
    info = plsc.get_sparse_core_info()
    nw = info.num_cores * info.num_subcores
    tpw = T // nw
    assert tpw % 8 == 0 and tpw <= 128
    mesh = plsc.VectorSubcoreMesh(core_axis_name="c", subcore_axis_name="s")

    @functools.partial(
        pl.kernel,
        mesh=mesh,
        out_type=jax.ShapeDtypeStruct((A_PAD, DM), jnp.float32),
        scratch_types=[
            pltpu.VMEM((tpw,), jnp.int32),
            pltpu.VMEM((tpw,), jnp.int32),
            pltpu.VMEM((tpw, DM), jnp.float32),
            pltpu.SemaphoreType.DMA,
            pltpu.SemaphoreType.DMA,
        ],
    )
    def k(table_hbm, pe_hbm, po_hbm, out_hbm, pe_v, po_v, rows_v, s1, s2):
        wid = lax.axis_index("s") * info.num_cores + lax.axis_index("c")
        base = wid * tpw
        pltpu.sync_copy(pe_hbm.at[pl.ds(base, tpw)], pe_v)
        pltpu.sync_copy(po_hbm.at[pl.ds(base, tpw)], po_v)
        pltpu.sync_copy(table_hbm.at[pl.ds(base, tpw)], rows_v)
        cp1 = pltpu.async_copy(rows_v, out_hbm.at[pe_v], s1)
        cp2 = pltpu.async_copy(rows_v, out_hbm.at[po_v], s2)
        cp1.wait()
        cp2.wait()

    return k


def _sc_dispatch(table, pe, po):
    return _make_sc_dispatch()(table, pe, po)


NT = A_PAD // TILE


def _pos_kernel(idx_ref, p_ref, gid_ref, oh_ref):
    e = idx_ref[...].reshape(A_RAW, 1)
    eids = lax.broadcasted_iota(jnp.int32, (A_RAW, E), 1)
    oh = (e == eids).astype(jnp.float32)
    oh_ref[...] = oh
    counts = jnp.sum(oh, 0, keepdims=True)
    pc = jnp.ceil(counts * (1.0 / TILE)) * TILE
    up = (lax.broadcasted_iota(jnp.int32, (E, E), 0) <
          lax.broadcasted_iota(jnp.int32, (E, E), 1)).astype(jnp.float32)
    pstart = jnp.dot(pc, up, preferred_element_type=jnp.float32)
    tril = (lax.broadcasted_iota(jnp.int32, (TILE, TILE), 1) <
            lax.broadcasted_iota(jnp.int32, (TILE, TILE), 0)).astype(jnp.float32)

    def body(i, carry):
        blk = oh_ref[pl.ds(i * TILE, TILE), :]
        rank = jnp.dot(tril, blk, preferred_element_type=jnp.float32) + carry
        pvals = jnp.sum(blk * (rank + pstart), 1, keepdims=True)
        p_ref[pl.ds(i * TILE, TILE), :] = pvals.astype(jnp.int32)
        return carry + jnp.sum(blk, 0, keepdims=True)

    lax.fori_loop(0, A_RAW // TILE, body, jnp.zeros((1, E), jnp.float32))

    trow = (lax.broadcasted_iota(jnp.int32, (E, NT), 1) * TILE).astype(
        jnp.float32)
    cmp = trow >= pstart.reshape(E, 1)
    gid_ref[...] = jnp.sum(cmp.astype(jnp.int32), 0, keepdims=True) - 1


def _positions(idx):
    return pl.pallas_call(
        _pos_kernel,
        grid=(1,),
        in_specs=[pl.BlockSpec((A_RAW, 1), lambda i: (0, 0))],
        out_specs=[
            pl.BlockSpec((A_RAW, 1), lambda i: (0, 0)),
            pl.BlockSpec((1, NT), lambda i: (0, 0)),
        ],
        out_shape=[
            jax.ShapeDtypeStruct((A_RAW, 1), jnp.int32),
            jax.ShapeDtypeStruct((1, NT), jnp.int32),
        ],
        scratch_shapes=[pltpu.VMEM((A_RAW, E), jnp.float32)],
    )(idx.reshape(A_RAW, 1))


def _cv_sq(v):
    return jnp.var(v) / (jnp.mean(v) ** 2 + 1e-10)


def _pos_enc():
    pos = jnp.arange(NPATCH, dtype=jnp.float32)[:, None]
    even = jnp.arange(0, DM, 2, dtype=jnp.float32)
    odd = jnp.arange(1, DM, 2, dtype=jnp.float32)
    enc = jnp.zeros((NPATCH, DM), jnp.float32)
    enc = enc.at[:, 0::2].set(jnp.sin(pos / jnp.power(10000.0, even / DM)))
    enc = enc.at[:, 1::2].set(jnp.cos(pos / jnp.power(10000.0, odd / DM)))
    return enc


def kernel(x, W_emb, b_emb, Wq, bq, Wk, bk, Wv, bv, Wo, bo, ln1_g, ln1_b,
           ln2_g, ln2_b, W_gate, We1, be1, We2, be2, W_fc, b_fc, W_out, b_out):
    x2 = x.reshape(S, B)
    xn, mu, sd = _instnorm(x2)
    patches = xn.reshape(NPATCH, DP, B).transpose(0, 2, 1).reshape(T, DP)
    pe_full = jnp.repeat(_pos_enc(), B, axis=0)
    h2 = _mm(patches, W_emb, bias=b_emb, res=pe_full)

    aux = jnp.asarray(0.0, jnp.float32)
    for l in range(NL):
        wqkv = jnp.concatenate([Wq[l], Wk[l], Wv[l]], axis=1)
        bqkv = jnp.concatenate([bq[l], bk[l], bv[l]], axis=0)
        qkv = _mm(h2, wqkv, bias=bqkv)
        qkv5 = qkv.reshape(NPATCH, B, 3, NH, D_HEAD).transpose(2, 1, 3, 0, 4)
        qkv5 = qkv5.reshape(3, B * NH, NPATCH, D_HEAD)
        o = _attention(qkv5[0], qkv5[1], qkv5[2])
        o = o.reshape(B, NH, NPATCH, D_HEAD).transpose(2, 0, 1, 3).reshape(T, DM)
        h2 = _mm(o, Wo[l], bias=bo[l], res=h2, ln=(ln1_g[l], ln1_b[l]))

        idx, w, imp, loads = _router(h2, W_gate[l])
        aux = aux + IMP_F * _cv_sq(imp[0]) + LOAD_F * _cv_sq(loads[0])

        p2, gid2 = _positions(idx)
        p = p2.reshape(A_RAW)
        tile_gid = gid2.reshape(NT)
        pe = p2.reshape(T, TK)[:, 0]
        po = p2.reshape(T, TK)[:, 1]
        xg = _sc_dispatch(h2, pe, po)
        y1 = _gm1(tile_gid, xg, We1[l], be1[l])
        y2 = _gm2(tile_gid, y1, We2[l], be2[l])
        yg = _sc_gather_combine(y2, p)
        h2 = _combine_ln(yg.reshape(T, TK, DM), w, h2, ln2_g[l], ln2_b[l])

    dxp = _mm(h2, W_fc, bias=b_fc, act="relu")
    dxb = dxp.reshape(NPATCH, B, DP).transpose(1, 0, 2).reshape(B, NPATCH * DP)
    rs = (sd + EPS).reshape(B, 1)
    ra = mu.reshape(B, 1)
    y = _mm(dxb, W_out, bias=b_out, row_affine=(rs, ra), bm=B)
    out = y.transpose(1, 0).reshape(S, B, 1)
    return out, aux
